# baseline (speedup 1.0000x reference)
import functools

import jax
import jax.numpy as jnp
from jax import lax
from jax.experimental import pallas as pl
from jax.experimental.pallas import tpu as pltpu
from jax.experimental.pallas import tpu_sc as plsc

_B = 1024
_M = 256
_D = 128

_info = plsc.get_sparse_core_info()
_NC, _NS, _L = _info.num_cores, _info.num_subcores, _info.num_lanes
_NW = _NC * _NS
_CH = _B // _NW

_CROWS = 128
_NBUF = 6
_PRE = 3
_QOUT = 3
_SLAB = _CH * _M
_NCHUNK = _SLAB // _CROWS
_HALF = _M // _CROWS

_mesh = plsc.VectorSubcoreMesh(core_axis_name="c", subcore_axis_name="s")


@functools.partial(
    pl.kernel,
    mesh=_mesh,
    out_type=[
        jax.ShapeDtypeStruct((_B * _M, _D), jnp.float32),
        jax.ShapeDtypeStruct((_B,), jnp.int32),
    ],
    scratch_types=[
        pltpu.VMEM((_NBUF, _CROWS, _D), jnp.float32),
        pltpu.VMEM((_CH + _L,), jnp.int32),
        pltpu.VMEM((_CH,), jnp.int32),
        pltpu.VMEM((_CH * _D,), jnp.float32),
        pltpu.SemaphoreType.DMA((_NBUF,)),
        pltpu.SemaphoreType.DMA((_NBUF,)),
    ],
)
def _sc_body(z_hbm, mem_hbm, state_hbm, out_hbm, ctr_hbm,
             bufs, state_v, ctr_v, z_v, sem_in, sem_out):
    wid = lax.axis_index("s") * _NC + lax.axis_index("c")
    base_b = wid * _CH
    row0 = base_b * _M

    def start_in(j, s):
        pltpu.make_async_copy(
            mem_hbm.at[pl.ds(row0 + j * _CROWS, _CROWS)],
            bufs.at[s], sem_in.at[s]).start()

    def wait_in(j, s):
        pltpu.make_async_copy(
            mem_hbm.at[pl.ds(row0 + j * _CROWS, _CROWS)],
            bufs.at[s], sem_in.at[s]).wait()

    def start_out(j, s):
        pltpu.make_async_copy(
            bufs.at[s], out_hbm.at[pl.ds(row0 + j * _CROWS, _CROWS)],
            sem_out.at[s]).start()

    def wait_out(j, s):
        pltpu.make_async_copy(
            bufs.at[s], out_hbm.at[pl.ds(row0 + j * _CROWS, _CROWS)],
            sem_out.at[s]).wait()

    def patch(j, s):
        b = j // _HALF
        r = lax.rem(state_v[pl.ds(b, _L)][0], _M)

        @pl.when(r // _CROWS == lax.rem(j, _HALF))
        def _():
            r_loc = lax.rem(r, _CROWS)
            for c in range(_D // _L):
                bufs[s, r_loc, pl.ds(c * _L, _L)] = (
                    z_v[pl.ds(b * _D + c * _L, _L)])

    pltpu.sync_copy(state_hbm.at[pl.ds(base_b, _CH)], state_v.at[pl.ds(0, _CH)])
    pltpu.sync_copy(z_hbm.at[pl.ds(base_b * _D, _CH * _D)], z_v)

    for c in range(_PRE):
        start_in(c, c % _NBUF)

    for j in range(_NBUF):
        s = j % _NBUF
        wait_in(j, s)
        patch(j, s)
        start_out(j, s)
        if j >= _QOUT:
            wait_out(j - _QOUT, (j - _QOUT) % _NBUF)
        start_in(j + _PRE, (j + _PRE) % _NBUF)

    _NGROUPS = _NCHUNK // _NBUF
    @pl.loop(1, _NGROUPS)
    def _ring(i):
        j0 = i * _NBUF
        for s in range(_NBUF):
            j = j0 + s
            wait_in(j, s)
            patch(j, s)
            start_out(j, s)
            wait_out(j - _QOUT, (j - _QOUT) % _NBUF)
            start_in(j + _PRE, (j + _PRE) % _NBUF)

    for j in range(_NGROUPS * _NBUF, _NCHUNK):
        s = j % _NBUF
        wait_in(j, s)
        patch(j, s)
        start_out(j, s)
        wait_out(j - _QOUT, (j - _QOUT) % _NBUF)
        if j + _PRE < _NCHUNK:
            start_in(j + _PRE, (j + _PRE) % _NBUF)

    for k in range(_CH // _L):
        sv = state_v[pl.ds(k * _L, _L)]
        ctr_v[pl.ds(k * _L, _L)] = sv + 1

    pltpu.sync_copy(ctr_v, ctr_hbm.at[pl.ds(base_b, _CH)])

    for q in range(_QOUT):
        j = _NCHUNK - _QOUT + q
        wait_out(j, j % _NBUF)


def kernel(z, mem_state, state):
    b, m, d = mem_state.shape
    mem2d = mem_state.reshape(b * m, d)
    out2d, ctr = _sc_body(z.reshape(b * d), mem2d, state)
    return out2d.reshape(b, m, d), ctr


# speedup vs baseline: 1.2864x; 1.2864x over previous
import jax
import jax.numpy as jnp
from jax import lax
from jax.experimental import pallas as pl
from jax.experimental.pallas import tpu as pltpu

_B = 1024
_M = 256
_D = 128
_NBUF = 12
_PRE = 6
_QOUT = 5

_SIZES = [4, 4, 8, 8, 16, 16] + [32] * 29 + [16, 8, 8, 4, 2, 2]
_STARTS = [sum(_SIZES[:i]) for i in range(len(_SIZES))]
_NCHUNK = len(_SIZES)
_CBMAX = max(_SIZES)


def _body(state_sref, state_ref, z_ref, mem_ref, out_ref, ctr_ref,
          bufs, sem_in, sem_out):
    ctr_ref[...] = state_ref[...] + 1

    def cp_in(j, s):
        nb = _SIZES[j]
        return pltpu.make_async_copy(
            mem_ref.at[pl.ds(_STARTS[j] * _M, nb * _M)],
            bufs.at[s].at[pl.ds(0, nb * _M)], sem_in.at[s])

    def cp_out(j, s):
        nb = _SIZES[j]
        return pltpu.make_async_copy(
            bufs.at[s].at[pl.ds(0, nb * _M)],
            out_ref.at[pl.ds(_STARTS[j] * _M, nb * _M)], sem_out.at[s])

    def patch(j, s):
        for b in range(_SIZES[j]):
            gb = _STARTS[j] + b
            r = lax.rem(state_sref[gb], _M)
            bufs[s, pl.ds(b * _M + r, 1), :] = z_ref[pl.ds(gb, 1), :]

    for c in range(_PRE):
        cp_in(c, c % _NBUF).start()

    for j in range(_NCHUNK):
        s = j % _NBUF
        cp_in(j, s).wait()
        patch(j, s)
        cp_out(j, s).start()
        if j >= _QOUT:
            jq = j - _QOUT
            cp_out(jq, jq % _NBUF).wait()
        if j + _PRE < _NCHUNK:
            jn = j + _PRE
            cp_in(jn, jn % _NBUF).start()

    for q in range(_QOUT):
        j = _NCHUNK - _QOUT + q
        cp_out(j, j % _NBUF).wait()


def kernel(z, mem_state, state):
    b, m, d = mem_state.shape
    mem2d = mem_state.reshape(b * m, d)
    state2d = state.reshape(b, 1)
    grid_spec = pltpu.PrefetchScalarGridSpec(
        num_scalar_prefetch=1,
        grid=(1,),
        in_specs=[
            pl.BlockSpec((b, 1), lambda i, s_ref: (0, 0)),
            pl.BlockSpec((b, d), lambda i, s_ref: (0, 0)),
            pl.BlockSpec(memory_space=pltpu.MemorySpace.HBM),
        ],
        out_specs=[
            pl.BlockSpec(memory_space=pltpu.MemorySpace.HBM),
            pl.BlockSpec((b, 1), lambda i, s_ref: (0, 0)),
        ],
        scratch_shapes=[
            pltpu.VMEM((_NBUF, _CBMAX * _M, _D), jnp.float32),
            pltpu.SemaphoreType.DMA((_NBUF,)),
            pltpu.SemaphoreType.DMA((_NBUF,)),
        ],
    )
    out2d, ctr2d = pl.pallas_call(
        _body,
        grid_spec=grid_spec,
        out_shape=[
            jax.ShapeDtypeStruct((b * m, d), mem_state.dtype),
            jax.ShapeDtypeStruct((b, 1), state.dtype),
        ],
    )(state, state2d, z, mem2d)
    return out2d.reshape(b, m, d), ctr2d.reshape(b)
